# Initial kernel scaffold; baseline (speedup 1.0000x reference)
import jax
import jax.numpy as jnp
from jax.experimental import pallas as pl


def kernel(z_1, z_2, dist_grade, img, given_param):
    raise NotImplementedError("write your pallas kernel here")


# speedup vs baseline: 1.2556x; 1.2556x over previous
import functools

import jax
import jax.numpy as jnp
import numpy as np
from jax import lax
from jax.experimental import pallas as pl
from jax.experimental.pallas import tpu as pltpu
from jax.experimental.pallas import tpu_sc as plsc

RHO_NUM = 100
B = 16384
D = 11

_NUM_WORKERS = 32
_ROWS_PER_W = B // _NUM_WORKERS
_GROUPS = _ROWS_PER_W // 16
_ZCHUNK = _ROWS_PER_W * D


def _bucket_thresholds():
    c = np.float32(1.0 / RHO_NUM)
    t = np.array([np.float32(0.01 * i) for i in range(1, RHO_NUM)], np.float32)

    def bucket_id(s):
        r = np.sqrt(np.float32(s), dtype=np.float32)
        k = np.round(np.float32(r / c)).astype(np.float32)
        return int(np.sum(np.float32(k * c) >= t))

    out = np.empty(RHO_NUM - 1, np.float32)
    for j in range(1, RHO_NUM):
        lo, hi = 0, int(np.array(1e8, np.float32).view(np.uint32))
        while lo < hi:
            mid = (lo + hi) // 2
            if bucket_id(np.array(mid, np.uint32).view(np.float32)) >= j:
                hi = mid
            else:
                lo = mid + 1
        out[j - 1] = np.array(lo, np.uint32).view(np.float32)
    return out


_S_LIST = [float(v) for v in _bucket_thresholds()]


def _sc_body(z1_hbm, z2_hbm, dg_hbm, sat_hbm, neq_hbm,
             z1v, z2v, idxv, maskv, valv, neqv, sem):
    wid = lax.axis_index("s") * 2 + lax.axis_index("c")
    zbase = wid * _ZCHUNK
    rbase = wid * _ROWS_PER_W

    pltpu.sync_copy(z1_hbm.at[pl.ds(zbase, _ZCHUNK)], z1v)
    pltpu.sync_copy(z2_hbm.at[pl.ds(zbase, _ZCHUNK)], z2v)

    lanes = lax.iota(jnp.int32, 16)

    def group(g, neq_acc):
        fbase = g * (16 * D)
        for u in range(D):
            a = z1v[pl.ds(fbase + u * 16, 16)]
            b = z2v[pl.ds(fbase + u * 16, 16)]
            neq_acc = jnp.where(a != b, 1.0, neq_acc)
        ridx = fbase + lanes * D
        z1_0 = plsc.load_gather(z1v, [ridx])
        z2_0 = plsc.load_gather(z2v, [ridx])
        z1_x = plsc.load_gather(z1v, [ridx + (D - 2)])
        z2_x = plsc.load_gather(z2v, [ridx + (D - 2)])
        z1_y = plsc.load_gather(z1v, [ridx + (D - 1)])
        z2_y = plsc.load_gather(z2v, [ridx + (D - 1)])
        dx = z1_x - z2_x
        dy = z1_y - z2_y
        s = dx * dx + dy * dy
        mf = jnp.where((z1_0 > 0.0) & (z2_0 > 0.0), 1.0, 0.0)
        bid = jnp.zeros((16,), jnp.int32)
        for thr in _S_LIST:
            bid = bid + (s >= thr).astype(jnp.int32)
        gidx = (rbase + g * 16 + lanes) * RHO_NUM + bid
        idxv[pl.ds(g * 16, 16)] = gidx
        maskv[pl.ds(g * 16, 16)] = mf
        return neq_acc

    neq_acc = lax.fori_loop(0, _GROUPS, group, jnp.zeros((16,), jnp.float32))
    neqv[...] = neq_acc
    pltpu.sync_copy(neqv, neq_hbm.at[wid])

    copies = [
        pltpu.async_copy(
            dg_hbm.at[idxv.at[pl.ds(i * 128, 128)]],
            valv.at[pl.ds(i * 128, 128)],
            sem,
        )
        for i in range(_ROWS_PER_W // 128)
    ]
    for c in copies:
        c.wait()

    for g in range(_GROUPS):
        sl = pl.ds(g * 16, 16)
        valv[sl] = valv[sl] * maskv[sl]
    pltpu.sync_copy(valv, sat_hbm.at[pl.ds(rbase, _ROWS_PER_W)])


_sc_fn = functools.partial(
    pl.kernel,
    mesh=plsc.VectorSubcoreMesh(core_axis_name="c", subcore_axis_name="s"),
    compiler_params=pltpu.CompilerParams(needs_layout_passes=False),
    out_type=[
        jax.ShapeDtypeStruct((B,), jnp.float32),
        jax.ShapeDtypeStruct((_NUM_WORKERS, 16), jnp.float32),
    ],
    scratch_types=[
        pltpu.VMEM((_ZCHUNK,), jnp.float32),
        pltpu.VMEM((_ZCHUNK,), jnp.float32),
        pltpu.VMEM((_ROWS_PER_W,), jnp.int32),
        pltpu.VMEM((_ROWS_PER_W,), jnp.float32),
        pltpu.VMEM((_ROWS_PER_W,), jnp.float32),
        pltpu.VMEM((16,), jnp.float32),
        pltpu.SemaphoreType.DMA,
    ],
)(_sc_body)


def _fix_body(sat_ref, neq_ref, out_ref):
    any_neq = jnp.max(neq_ref[...])
    out_ref[...] = jnp.where(any_neq > 0.0, sat_ref[...], 0.0)


@jax.jit
def _fix(sat2d, neq2d):
    return pl.pallas_call(
        _fix_body,
        out_shape=jax.ShapeDtypeStruct(sat2d.shape, jnp.float32),
    )(sat2d, neq2d)


def kernel(z_1, z_2, dist_grade, img, given_param):
    sat, neq = _sc_fn(
        z_1.reshape(-1), z_2.reshape(-1), dist_grade.reshape(-1)
    )
    out = _fix(sat.reshape(128, 128), neq.reshape(4, 128))
    return out.reshape(B)
